# baseline (speedup 1.0000x reference)
import functools

import jax
import jax.numpy as jnp
from jax import lax
from jax.experimental import pallas as pl
from jax.experimental.pallas import tpu as pltpu
from jax.experimental.pallas import tpu_sc as plsc

_NC = 2
_NS = 16
_NW = _NC * _NS
_L = 16


def _iota16():
    return lax.iota(jnp.int32, _L)


def _splat(v):
    return jnp.full((_L,), v, jnp.int32)


@functools.cache
def _build_transpose(V, D):
    NBLK = V // 128
    TAIL = V - NBLK * 128
    per_w = NBLK // _NW
    extra = NBLK - per_w * _NW
    mesh = plsc.VectorSubcoreMesh(core_axis_name="c", subcore_axis_name="s")

    @functools.partial(
        pl.kernel,
        out_type=jax.ShapeDtypeStruct((V * D,), jnp.float32),
        mesh=mesh,
        compiler_params=pltpu.CompilerParams(use_tc_tiling_on_sc=True,
                                             needs_layout_passes=False),
        scratch_types=[
            pltpu.VMEM((D, 128), jnp.float32),
            pltpu.VMEM((128 * D,), jnp.float32),
            pltpu.VMEM((max(TAIL * D, _L),), jnp.float32),
        ],
    )
    def ktrans(tblT_hbm, tail_hbm, out_hbm, stripe_v, rows_v, tail_v):
        wid = lax.axis_index("s") * _NC + lax.axis_index("c")
        base = [_splat((c * _L)) + _iota16() for c in range(D // _L)]

        def blk_body(j, carry):
            k = wid + j * _NW

            @pl.when(k < NBLK)
            def _():
                pltpu.sync_copy(tblT_hbm.at[:, pl.ds(k * 128, 128)], stripe_v)

                def row_body(i, c2):
                    col = _splat(i)
                    for c in range(D // _L):
                        v = plsc.load_gather(stripe_v, [base[c], col])
                        rows_v[pl.ds(i * D + c * _L, _L)] = v
                    return c2

                lax.fori_loop(0, 128, row_body, 0)
                pltpu.sync_copy(rows_v, out_hbm.at[pl.ds(k * 128 * D, 128 * D)])

            return carry

        lax.fori_loop(0, per_w + (1 if extra else 0), blk_body, 0)

        if TAIL:
            @pl.when(wid == _NW - 1)
            def _():
                pltpu.sync_copy(tail_hbm, tail_v.at[pl.ds(0, TAIL * D)])
                pltpu.sync_copy(tail_v.at[pl.ds(0, TAIL * D)],
                                out_hbm.at[pl.ds(NBLK * 128 * D, TAIL * D)])

    return ktrans


@functools.cache
def _build_gather(B, N, D, V):
    BB = B // 128
    assert BB == _NW, "worker split assumes B == 128 * 32"
    mesh = plsc.VectorSubcoreMesh(core_axis_name="c", subcore_axis_name="s")

    @functools.partial(
        pl.kernel,
        out_type=jax.ShapeDtypeStruct((N, D // 8, BB, 8, 128), jnp.float32),
        mesh=mesh,
        compiler_params=pltpu.CompilerParams(use_tc_tiling_on_sc=False,
                                             needs_layout_passes=False),
        scratch_types=[
            pltpu.VMEM((128,), jnp.int32),
            pltpu.VMEM((128, D), jnp.float32),
            pltpu.VMEM((D, 128), jnp.float32),
            pltpu.VMEM((N, D), jnp.float32),
            pltpu.SemaphoreType.DMA,
        ],
    )
    def kgath(idx_hbm, tbl_hbm, pos_hbm, out_hbm, idx_v, rows_v, obuf_v,
              pos_v, sem):
        wid = lax.axis_index("s") * _NC + lax.axis_index("c")
        pltpu.sync_copy(pos_hbm, pos_v)
        rowsel = [_splat(g * _L) + _iota16() for g in range(8)]

        def t_body(t, carry):
            pltpu.sync_copy(idx_hbm.at[pl.ds(t * B + wid * 128, 128)], idx_v)
            pltpu.async_copy(tbl_hbm.at[idx_v], rows_v, sem).wait()
            tcol = _splat(t)

            def d_body(d, c2):
                dcol = _splat(d)
                p = plsc.load_gather(pos_v, [tcol, dcol])
                for g in range(8):
                    v = plsc.load_gather(rows_v, [rowsel[g], dcol])
                    obuf_v[d, pl.ds(g * _L, _L)] = v + p
                return c2

            lax.fori_loop(0, D, d_body, 0)
            for dt in range(D // 8):
                pltpu.sync_copy(obuf_v.at[pl.ds(dt * 8, 8), :],
                                out_hbm.at[t, dt, wid])
            return carry

        lax.fori_loop(0, N, t_body, 0)

    return kgath


def kernel(x, table, pos_embedding):
    B, N = x.shape
    V, D = table.shape
    NBLK = V // 128
    tail = table[NBLK * 128:, :].reshape(-1)
    tbl_lin = _build_transpose(V, D)(table.T, tail)
    idxT = x.T.reshape(-1).astype(jnp.int32)
    out5 = _build_gather(B, N, D, V)(idxT, tbl_lin.reshape(V, D),
                                     pos_embedding)
    return out5.transpose(2, 4, 0, 1, 3).reshape(B, N, D)


# speedup vs baseline: 1.2737x; 1.2737x over previous
import functools

import jax
import jax.numpy as jnp
from jax import lax
from jax.experimental import pallas as pl
from jax.experimental.pallas import tpu as pltpu
from jax.experimental.pallas import tpu_sc as plsc

_NC = 2
_NS = 16
_NW = _NC * _NS
_L = 16


def _iota16():
    return lax.iota(jnp.int32, _L)


def _splat(v):
    return jnp.full((_L,), v, jnp.int32)


@functools.cache
def _build_transpose(V, D):
    NBLK = V // 128
    TAIL = V - NBLK * 128
    per_w = NBLK // _NW
    rem = NBLK - per_w * _NW
    mesh = plsc.VectorSubcoreMesh(core_axis_name="c", subcore_axis_name="s")

    @functools.partial(
        pl.kernel,
        out_type=jax.ShapeDtypeStruct((V * D,), jnp.float32),
        mesh=mesh,
        compiler_params=pltpu.CompilerParams(use_tc_tiling_on_sc=True,
                                             needs_layout_passes=False),
        scratch_types=[
            pltpu.VMEM((D, 128), jnp.float32),
            pltpu.VMEM((D, 128), jnp.float32),
            pltpu.VMEM((128 * D,), jnp.float32),
            pltpu.VMEM((128 * D,), jnp.float32),
            pltpu.VMEM((max(TAIL * D, _L),), jnp.float32),
            pltpu.SemaphoreType.DMA,
            pltpu.SemaphoreType.DMA,
            pltpu.SemaphoreType.DMA,
            pltpu.SemaphoreType.DMA,
        ],
    )
    def ktrans(tblT_hbm, tail_hbm, out_hbm, s0, s1, r0, r1, tail_v,
               gi0, gi1, go0, go1):
        wid = lax.axis_index("s") * _NC + lax.axis_index("c")
        base = [_splat(c * _L) + _iota16() for c in range(D // _L)]
        stripes = (s0, s1)
        rows = (r0, r1)
        gis = (gi0, gi1)
        gos = (go0, go1)

        def blk(w, j):
            return wid + (j * 2 + w) * _NW

        def fire_in(w, j):
            k = blk(w, j)
            pltpu.async_copy(tblT_hbm.at[:, pl.ds(k * 128, 128)],
                             stripes[w], gis[w])

        def transpose_block(w, j):
            k = blk(w, j)
            sv, rv = stripes[w], rows[w]

            def row_body(i, c2):
                col = _splat(i)
                for c in range(D // _L):
                    v = plsc.load_gather(sv, [base[c], col])
                    rv[pl.ds(i * D + c * _L, _L)] = v
                return c2

            lax.fori_loop(0, 128, row_body, 0, unroll=4)
            pltpu.async_copy(rv, out_hbm.at[pl.ds(k * 128 * D, 128 * D)],
                             gos[w])

        def wait_in(w):
            pltpu.make_async_copy(tblT_hbm.at[:, pl.ds(0, 128)],
                                  stripes[w], gis[w]).wait()

        def wait_out(w):
            pltpu.make_async_copy(rows[w],
                                  out_hbm.at[pl.ds(0, 128 * D)],
                                  gos[w]).wait()

        fire_in(0, 0)

        def pair_body(j, carry):
            fire_in(1, j)
            wait_in(0)

            @pl.when(j > 0)
            def _():
                wait_out(0)

            transpose_block(0, j)

            @pl.when(j + 1 < per_w // 2)
            def _():
                fire_in(0, j + 1)

            wait_in(1)

            @pl.when(j > 0)
            def _():
                wait_out(1)

            transpose_block(1, j)
            return carry

        lax.fori_loop(0, per_w // 2, pair_body, 0)
        wait_out(0)
        wait_out(1)

        if rem:
            @pl.when(wid < rem)
            def _():
                k = _NW * per_w + wid
                pltpu.sync_copy(tblT_hbm.at[:, pl.ds(k * 128, 128)], s0)

                def row_body(i, c2):
                    col = _splat(i)
                    for c in range(D // _L):
                        v = plsc.load_gather(s0, [base[c], col])
                        r0[pl.ds(i * D + c * _L, _L)] = v
                    return c2

                lax.fori_loop(0, 128, row_body, 0, unroll=4)
                pltpu.sync_copy(r0, out_hbm.at[pl.ds(k * 128 * D, 128 * D)])

        if TAIL:
            @pl.when(wid == _NW - 1)
            def _():
                pltpu.sync_copy(tail_hbm, tail_v.at[pl.ds(0, TAIL * D)])
                pltpu.sync_copy(tail_v.at[pl.ds(0, TAIL * D)],
                                out_hbm.at[pl.ds(NBLK * 128 * D, TAIL * D)])

    return ktrans


@functools.cache
def _build_gather(B, N, D, V):
    BB = B // 128
    assert BB == _NW, "worker split assumes B == 128 * 32"
    mesh = plsc.VectorSubcoreMesh(core_axis_name="c", subcore_axis_name="s")

    @functools.partial(
        pl.kernel,
        out_type=jax.ShapeDtypeStruct((N, D // 8, BB, 8, 128), jnp.float32),
        mesh=mesh,
        compiler_params=pltpu.CompilerParams(use_tc_tiling_on_sc=False,
                                             needs_layout_passes=False),
        scratch_types=[
            pltpu.VMEM((N, 128), jnp.int32),
            pltpu.VMEM((128, D), jnp.float32),
            pltpu.VMEM((128, D), jnp.float32),
            pltpu.VMEM((D // 8, 8, 128), jnp.float32),
            pltpu.VMEM((D // 8, 8, 128), jnp.float32),
            pltpu.VMEM((N, D), jnp.float32),
            pltpu.SemaphoreType.DMA,
            pltpu.SemaphoreType.DMA,
            pltpu.SemaphoreType.DMA,
            pltpu.SemaphoreType.DMA,
        ],
    )
    def kgath(idx_hbm, tbl_hbm, pos_hbm, out_hbm, idx_v, ra, rb, oa, ob,
              pos_v, ga, gb, sa, sb):
        wid = lax.axis_index("s") * _NC + lax.axis_index("c")
        pltpu.sync_copy(pos_hbm, pos_v)
        pltpu.sync_copy(idx_hbm.at[:, wid, :], idx_v)
        rowsel = [_splat(g * _L) + _iota16() for g in range(8)]
        rows = (ra, rb)
        obufs = (oa, ob)
        gsems = (ga, gb)
        osems = (sa, sb)

        def fire_gather(w, t):
            pltpu.async_copy(tbl_hbm.at[idx_v.at[t]], rows[w], gsems[w])

        def wait_gather(w):
            pltpu.make_async_copy(tbl_hbm.at[idx_v.at[0]], rows[w],
                                  gsems[w]).wait()

        def wait_out(w):
            pltpu.make_async_copy(obufs[w], out_hbm.at[0, :, wid],
                                  osems[w]).wait()

        def compute(w, t):
            rv, ov = rows[w], obufs[w]
            tcol = _splat(t)

            def d_body(d, c2):
                dcol = _splat(d)
                p = plsc.load_gather(pos_v, [tcol, dcol])
                dt = d // 8
                ds_ = d % 8
                for g in range(8):
                    v = plsc.load_gather(rv, [rowsel[g], dcol])
                    ov[dt, ds_, pl.ds(g * _L, _L)] = v + p
                return c2

            lax.fori_loop(0, D, d_body, 0, unroll=2)
            pltpu.async_copy(ov, out_hbm.at[t, :, wid], osems[w])

        fire_gather(0, 0)

        def pair_body(j, carry):
            tA = j * 2
            fire_gather(1, tA + 1)
            wait_gather(0)

            @pl.when(j > 0)
            def _():
                wait_out(0)

            compute(0, tA)

            @pl.when(j + 1 < N // 2)
            def _():
                fire_gather(0, tA + 2)

            wait_gather(1)

            @pl.when(j > 0)
            def _():
                wait_out(1)

            compute(1, tA + 1)
            return carry

        lax.fori_loop(0, N // 2, pair_body, 0)
        wait_out(0)
        wait_out(1)

    return kgath


def kernel(x, table, pos_embedding):
    B, N = x.shape
    V, D = table.shape
    NBLK = V // 128
    tail = table[NBLK * 128:, :].reshape(-1)
    tbl_lin = _build_transpose(V, D)(table.T, tail)
    idx3 = x.T.reshape(N, B // 128, 128).astype(jnp.int32)
    out5 = _build_gather(B, N, D, V)(idx3, tbl_lin.reshape(V, D),
                                     pos_embedding)
    return out5.transpose(2, 4, 0, 1, 3).reshape(B, N, D)


# speedup vs baseline: 2.4485x; 1.9224x over previous
import functools

import jax
import jax.numpy as jnp
from jax import lax
from jax.experimental import pallas as pl
from jax.experimental.pallas import tpu as pltpu
from jax.experimental.pallas import tpu_sc as plsc

_NC = 2
_NS = 16
_NW = _NC * _NS
_L = 16


def _iota16():
    return lax.iota(jnp.int32, _L)


def _splat(v):
    return jnp.full((_L,), v, jnp.int32)


@functools.cache
def _build_transpose(V, D):
    NBLK = V // 128
    TAIL = V - NBLK * 128
    per_w = NBLK // _NW
    rem = NBLK - per_w * _NW
    mesh = plsc.VectorSubcoreMesh(core_axis_name="c", subcore_axis_name="s")

    @functools.partial(
        pl.kernel,
        out_type=jax.ShapeDtypeStruct((V * D,), jnp.float32),
        mesh=mesh,
        compiler_params=pltpu.CompilerParams(use_tc_tiling_on_sc=True,
                                             needs_layout_passes=False),
        scratch_types=[
            pltpu.VMEM((D, 128), jnp.float32),
            pltpu.VMEM((D, 128), jnp.float32),
            pltpu.VMEM((128 * D,), jnp.float32),
            pltpu.VMEM((128 * D,), jnp.float32),
            pltpu.VMEM((max(TAIL * D, _L),), jnp.float32),
            pltpu.SemaphoreType.DMA,
            pltpu.SemaphoreType.DMA,
            pltpu.SemaphoreType.DMA,
            pltpu.SemaphoreType.DMA,
        ],
    )
    def ktrans(tblT_hbm, tail_hbm, out_hbm, s0, s1, r0, r1, tail_v,
               gi0, gi1, go0, go1):
        wid = lax.axis_index("s") * _NC + lax.axis_index("c")
        base = [_splat(c * _L) + _iota16() for c in range(D // _L)]
        stripes = (s0, s1)
        rows = (r0, r1)
        gis = (gi0, gi1)
        gos = (go0, go1)

        def blk(w, j):
            return wid + (j * 2 + w) * _NW

        def fire_in(w, j):
            k = blk(w, j)
            pltpu.async_copy(tblT_hbm.at[:, pl.ds(k * 128, 128)],
                             stripes[w], gis[w])

        def transpose_block(w, j):
            k = blk(w, j)
            sv, rv = stripes[w], rows[w]

            @plsc.parallel_loop(0, 128, unroll=8)
            def row_body(i):
                col = _splat(i)
                for c in range(D // _L):
                    v = plsc.load_gather(sv, [base[c], col])
                    rv[pl.ds(i * D + c * _L, _L)] = v
            pltpu.async_copy(rv, out_hbm.at[pl.ds(k * 128 * D, 128 * D)],
                             gos[w])

        def wait_in(w):
            pltpu.make_async_copy(tblT_hbm.at[:, pl.ds(0, 128)],
                                  stripes[w], gis[w]).wait()

        def wait_out(w):
            pltpu.make_async_copy(rows[w],
                                  out_hbm.at[pl.ds(0, 128 * D)],
                                  gos[w]).wait()

        fire_in(0, 0)

        def pair_body(j, carry):
            fire_in(1, j)
            wait_in(0)

            @pl.when(j > 0)
            def _():
                wait_out(0)

            transpose_block(0, j)

            @pl.when(j + 1 < per_w // 2)
            def _():
                fire_in(0, j + 1)

            wait_in(1)

            @pl.when(j > 0)
            def _():
                wait_out(1)

            transpose_block(1, j)
            return carry

        lax.fori_loop(0, per_w // 2, pair_body, 0)
        wait_out(0)
        wait_out(1)

        if rem:
            @pl.when(wid < rem)
            def _():
                k = _NW * per_w + wid
                pltpu.sync_copy(tblT_hbm.at[:, pl.ds(k * 128, 128)], s0)

                @plsc.parallel_loop(0, 128, unroll=8)
                def row_body(i):
                    col = _splat(i)
                    for c in range(D // _L):
                        v = plsc.load_gather(s0, [base[c], col])
                        r0[pl.ds(i * D + c * _L, _L)] = v
                pltpu.sync_copy(r0, out_hbm.at[pl.ds(k * 128 * D, 128 * D)])

        if TAIL:
            @pl.when(wid == _NW - 1)
            def _():
                pltpu.sync_copy(tail_hbm, tail_v.at[pl.ds(0, TAIL * D)])
                pltpu.sync_copy(tail_v.at[pl.ds(0, TAIL * D)],
                                out_hbm.at[pl.ds(NBLK * 128 * D, TAIL * D)])

    return ktrans


@functools.cache
def _build_gather(B, N, D, V):
    BB = B // 128
    assert BB == _NW, "worker split assumes B == 128 * 32"
    mesh = plsc.VectorSubcoreMesh(core_axis_name="c", subcore_axis_name="s")

    @functools.partial(
        pl.kernel,
        out_type=jax.ShapeDtypeStruct((N, D // 8, BB, 8, 128), jnp.float32),
        mesh=mesh,
        compiler_params=pltpu.CompilerParams(use_tc_tiling_on_sc=False,
                                             needs_layout_passes=False),
        scratch_types=[
            pltpu.VMEM((N, 128), jnp.int32),
            pltpu.VMEM((128, D), jnp.float32),
            pltpu.VMEM((128, D), jnp.float32),
            pltpu.VMEM((D // 8, 8, 128), jnp.float32),
            pltpu.VMEM((D // 8, 8, 128), jnp.float32),
            pltpu.VMEM((N, D), jnp.float32),
            pltpu.SemaphoreType.DMA,
            pltpu.SemaphoreType.DMA,
            pltpu.SemaphoreType.DMA,
            pltpu.SemaphoreType.DMA,
        ],
    )
    def kgath(idx_hbm, tbl_hbm, pos_hbm, out_hbm, idx_v, ra, rb, oa, ob,
              pos_v, ga, gb, sa, sb):
        wid = lax.axis_index("s") * _NC + lax.axis_index("c")
        pltpu.sync_copy(pos_hbm, pos_v)
        pltpu.sync_copy(idx_hbm.at[:, wid, :], idx_v)
        rowsel = [_splat(g * _L) + _iota16() for g in range(8)]
        rows = (ra, rb)
        obufs = (oa, ob)
        gsems = (ga, gb)
        osems = (sa, sb)

        def fire_gather(w, t):
            pltpu.async_copy(tbl_hbm.at[idx_v.at[t]], rows[w], gsems[w])

        def wait_gather(w):
            pltpu.make_async_copy(tbl_hbm.at[idx_v.at[0]], rows[w],
                                  gsems[w]).wait()

        def wait_out(w):
            pltpu.make_async_copy(obufs[w], out_hbm.at[0, :, wid],
                                  osems[w]).wait()

        def compute(w, t):
            rv, ov = rows[w], obufs[w]
            tcol = _splat(t)

            @plsc.parallel_loop(0, D, unroll=4)
            def d_body(d):
                dcol = _splat(d)
                p = plsc.load_gather(pos_v, [tcol, dcol])
                dt = d // 8
                ds_ = d % 8
                for g in range(8):
                    v = plsc.load_gather(rv, [rowsel[g], dcol])
                    ov[dt, ds_, pl.ds(g * _L, _L)] = v + p
            pltpu.async_copy(ov, out_hbm.at[t, :, wid], osems[w])

        fire_gather(0, 0)

        def pair_body(j, carry):
            tA = j * 2
            fire_gather(1, tA + 1)
            wait_gather(0)

            @pl.when(j > 0)
            def _():
                wait_out(0)

            compute(0, tA)

            @pl.when(j + 1 < N // 2)
            def _():
                fire_gather(0, tA + 2)

            wait_gather(1)

            @pl.when(j > 0)
            def _():
                wait_out(1)

            compute(1, tA + 1)
            return carry

        lax.fori_loop(0, N // 2, pair_body, 0)
        wait_out(0)
        wait_out(1)

    return kgath


def kernel(x, table, pos_embedding):
    B, N = x.shape
    V, D = table.shape
    NBLK = V // 128
    tail = table[NBLK * 128:, :].reshape(-1)
    tbl_lin = _build_transpose(V, D)(table.T, tail)
    idx3 = x.T.reshape(N, B // 128, 128).astype(jnp.int32)
    out5 = _build_gather(B, N, D, V)(idx3, tbl_lin.reshape(V, D),
                                     pos_embedding)
    return out5.transpose(2, 4, 0, 1, 3).reshape(B, N, D)
